# baseline (speedup 1.0000x reference)
import functools

import jax
import jax.numpy as jnp
from jax import lax
from jax.experimental import pallas as pl
from jax.experimental.pallas import tpu as pltpu
from jax.experimental.pallas import tpu_sc as plsc

N = 10000
NP = 10240
E = 320000
EP = 327680
ER = EP // 128
NC = 2
NS = 16
NW = NC * NS
RT = ER // NW
ETILE = EP // NW
ET_REAL = E // NW
NT = NP // NS
HG = 16
F32 = jnp.float32


def _node_proj_body(x_ref, w_ref, as_ref, ad_ref, h_ref, als_ref, ald_ref):
    h = jnp.dot(x_ref[...], w_ref[...], preferred_element_type=F32)
    h_ref[...] = h
    als_ref[...] = h @ as_ref[...]
    ald_ref[...] = h @ ad_ref[...]


def _node_proj(x, w, a_s, a_d):
    nb = 5
    blk = NP // nb
    d_in = x.shape[1]
    return pl.pallas_call(
        _node_proj_body,
        grid=(nb,),
        in_specs=[
            pl.BlockSpec((blk, d_in), lambda i: (i, 0)),
            pl.BlockSpec((d_in, HG), lambda i: (0, 0)),
            pl.BlockSpec((HG,), lambda i: (0,)),
            pl.BlockSpec((HG,), lambda i: (0,)),
        ],
        out_specs=[
            pl.BlockSpec((blk, HG), lambda i: (i, 0)),
            pl.BlockSpec((blk,), lambda i: (i,)),
            pl.BlockSpec((blk,), lambda i: (i,)),
        ],
        out_shape=[
            jax.ShapeDtypeStruct((NP, HG), F32),
            jax.ShapeDtypeStruct((NP,), F32),
            jax.ShapeDtypeStruct((NP,), F32),
        ],
    )(x, w, a_s, a_d)


def _ale_body(ea_ref, we_ref, ae_ref, o0_ref, o1_ref, o2_ref):
    ea = ea_ref[...]
    for i, o_ref in enumerate((o0_ref, o1_ref, o2_ref)):
        v = jnp.dot(we_ref[i], ae_ref[i], preferred_element_type=F32)
        o_ref[...] = ea @ v


def _edge_logits(edge_attr_p, we_all, ae_all):
    nb = 80
    blk = EP // nb
    d_e = edge_attr_p.shape[1]
    return pl.pallas_call(
        _ale_body,
        grid=(nb,),
        in_specs=[
            pl.BlockSpec((blk, d_e), lambda i: (i, 0)),
            pl.BlockSpec((3, d_e, HG), lambda i: (0, 0, 0)),
            pl.BlockSpec((3, HG), lambda i: (0, 0)),
        ],
        out_specs=[pl.BlockSpec((blk,), lambda i: (i,))] * 3,
        out_shape=[jax.ShapeDtypeStruct((EP,), F32)] * 3,
    )(edge_attr_p, we_all, ae_all)


def _combine_proj_body(num_ref, den_ref, b_ref, w_ref, as_ref, ad_ref,
                       h_ref, als_ref, ald_ref):
    num = num_ref[0] + num_ref[1]
    den = den_ref[0] + den_ref[1]
    xn = num / (den[:, None] + 1e-16) + b_ref[...]
    xn = jnp.where(xn >= 0.0, xn, 0.2 * xn)
    h = jnp.dot(xn, w_ref[...], preferred_element_type=F32)
    h_ref[...] = h
    als_ref[...] = h @ as_ref[...]
    ald_ref[...] = h @ ad_ref[...]


def _combine_proj(num, den, b, w, a_s, a_d):
    nb = 5
    blk = NP // nb
    return pl.pallas_call(
        _combine_proj_body,
        grid=(nb,),
        in_specs=[
            pl.BlockSpec((NC, blk, HG), lambda i: (0, i, 0)),
            pl.BlockSpec((NC, blk), lambda i: (0, i)),
            pl.BlockSpec((HG,), lambda i: (0,)),
            pl.BlockSpec((HG, HG), lambda i: (0, 0)),
            pl.BlockSpec((HG,), lambda i: (0,)),
            pl.BlockSpec((HG,), lambda i: (0,)),
        ],
        out_specs=[
            pl.BlockSpec((blk, HG), lambda i: (i, 0)),
            pl.BlockSpec((blk,), lambda i: (i,)),
            pl.BlockSpec((blk,), lambda i: (i,)),
        ],
        out_shape=[
            jax.ShapeDtypeStruct((NP, HG), F32),
            jax.ShapeDtypeStruct((NP,), F32),
            jax.ShapeDtypeStruct((NP,), F32),
        ],
    )(num, den, b, w, a_s, a_d)


def _mlp_body(num_ref, den_ref, b_ref, w0_ref, b0_ref, w1_ref, b1_ref,
              w2_ref, b2_ref, w3_ref, b3_ref, o_ref):
    num = num_ref[0] + num_ref[1]
    den = den_ref[0] + den_ref[1]
    a = num / (den[:, None] + 1e-16) + b_ref[...]
    a = jnp.where(a >= 0.0, a, 0.2 * a)
    for w_r, bias_r in ((w0_ref, b0_ref), (w1_ref, b1_ref), (w2_ref, b2_ref)):
        a = jnp.maximum(
            jnp.dot(a, w_r[...], preferred_element_type=F32) + bias_r[...], 0.0)
    o_ref[...] = jnp.dot(a, w3_ref[...], preferred_element_type=F32) + b3_ref[...]


def _mlp(num, den, b, l0w, l0b, l1w, l1b, l2w, l2b, l3w, l3b):
    nb = 5
    blk = NP // nb
    d_out = l3w.shape[1]
    hl = l0w.shape[1]
    return pl.pallas_call(
        _mlp_body,
        grid=(nb,),
        in_specs=[
            pl.BlockSpec((NC, blk, HG), lambda i: (0, i, 0)),
            pl.BlockSpec((NC, blk), lambda i: (0, i)),
            pl.BlockSpec((HG,), lambda i: (0,)),
            pl.BlockSpec((HG, hl), lambda i: (0, 0)),
            pl.BlockSpec((hl,), lambda i: (0,)),
            pl.BlockSpec((hl, hl), lambda i: (0, 0)),
            pl.BlockSpec((hl,), lambda i: (0,)),
            pl.BlockSpec((hl, hl), lambda i: (0, 0)),
            pl.BlockSpec((hl,), lambda i: (0,)),
            pl.BlockSpec((hl, d_out), lambda i: (0, 0)),
            pl.BlockSpec((d_out,), lambda i: (0,)),
        ],
        out_specs=[pl.BlockSpec((blk, d_out), lambda i: (i, 0))],
        out_shape=[jax.ShapeDtypeStruct((NP, d_out), F32)],
    )(num, den, b, l0w, l0b, l1w, l1b, l2w, l2b, l3w, l3b)[0]


def _sc_edge_layer(als, ald, h, ale, src, dst):
    mesh = plsc.VectorSubcoreMesh(core_axis_name="c", subcore_axis_name="s")
    WR = 2
    W = WR * 128
    NWIN = RT // WR
    _dn = lax.GatherDimensionNumbers(
        offset_dims=(), collapsed_slice_dims=(0,), start_index_map=(0,))

    @functools.partial(
        pl.kernel,
        out_type=[
            jax.ShapeDtypeStruct((NC, NP, HG), F32),
            jax.ShapeDtypeStruct((NC, NP), F32),
        ],
        mesh=mesh,
        scratch_types=[
            pltpu.VMEM((NP,), F32),
            pltpu.VMEM((NP,), F32),
            pltpu.VMEM((ETILE,), jnp.int32),
            pltpu.VMEM((ETILE,), jnp.int32),
            pltpu.VMEM((ETILE,), F32),
            pltpu.VMEM((ETILE,), F32),
            pltpu.VMEM((W, HG), F32),
            pltpu.VMEM((W, HG), F32),
            pltpu.VMEM((W, HG), F32),
            pltpu.VMEM((W, HG), F32),
            pltpu.VMEM((NT, HG), F32),
            pltpu.VMEM((NT,), F32),
            pltpu.VMEM_SHARED((NP, HG), F32),
            pltpu.VMEM_SHARED((NP, HG), F32),
            pltpu.VMEM_SHARED((NP,), F32),
            pltpu.SemaphoreType.DMA,
            pltpu.SemaphoreType.DMA,
            pltpu.SemaphoreType.DMA,
            pltpu.SemaphoreType.DMA,
        ],
        compiler_params=pltpu.CompilerParams(needs_layout_passes=False,
                                             use_tc_tiling_on_sc=False),
    )
    def body(als_h, ald_h, h_h, ale_h, src_h, dst_h, num_o, den_o,
             als_v, ald_v, src_v, dst_v, ale_v, ea_v,
             gbuf0, gbuf1, sbuf0, sbuf1, znum_v, zden_v,
             h_s, num_s, den_s, gsem0, gsem1, ssem0, ssem1):
        c = lax.axis_index("c")
        s = lax.axis_index("s")
        wid = s * NC + c
        bufs = ((gbuf0, sbuf0, gsem0, ssem0), (gbuf1, sbuf1, gsem1, ssem1))

        pltpu.sync_copy(als_h, als_v)
        pltpu.sync_copy(ald_h, ald_v)
        pltpu.sync_copy(h_h.at[pl.ds(s * NT, NT)], h_s.at[pl.ds(s * NT, NT)])
        zv = jnp.zeros((16,), F32)

        def zrow(j, _):
            znum_v[j, :] = zv
            return 0

        lax.fori_loop(0, NT, zrow, 0)

        def zden(j, _):
            zden_v[pl.ds(j * 16, 16)] = zv
            return 0

        lax.fori_loop(0, NT // 16, zden, 0)
        pltpu.sync_copy(znum_v, num_s.at[pl.ds(s * NT, NT)])
        pltpu.sync_copy(zden_v, den_s.at[pl.ds(s * NT, NT)])

        pltpu.sync_copy(src_h.at[pl.ds(wid * ET_REAL, ET_REAL)],
                        src_v.at[pl.ds(0, ET_REAL)])
        pltpu.sync_copy(dst_h.at[pl.ds(wid * ET_REAL, ET_REAL)],
                        dst_v.at[pl.ds(0, ET_REAL)])
        pltpu.sync_copy(ale_h.at[pl.ds(wid * ET_REAL, ET_REAL)],
                        ale_v.at[pl.ds(0, ET_REAL)])
        iv16 = lax.broadcasted_iota(jnp.int32, (16,), 0)

        def ptail(j, _):
            sl = pl.ds(ET_REAL + j * 16, 16)
            pad_ix = (iv16 * 977 + j * 131) % N
            src_v[sl] = pad_ix
            dst_v[sl] = pad_ix
            ale_v[sl] = jnp.full((16,), -1e30, F32)
            return 0

        lax.fori_loop(0, (ETILE - ET_REAL) // 16, ptail, 0)
        plsc.subcore_barrier()

        def g_desc(w, b):
            gbuf_, _, gsem_, _ = bufs[b]
            return pltpu.make_async_copy(
                h_s.at[src_v.at[pl.ds(w * W, W)]], gbuf_, gsem_)

        def s_desc(w, b):
            _, sbuf_, _, ssem_ = bufs[b]
            return pltpu.make_async_copy(
                sbuf_, num_s.at[dst_v.at[pl.ds(w * W, W)]], ssem_)

        pltpu.async_copy(h_s.at[src_v.at[pl.ds(0, W)]], gbuf0, gsem0)
        pltpu.async_copy(h_s.at[src_v.at[pl.ds(W, W)]], gbuf1, gsem1)

        def win(wo, _):
            for b in range(2):
                gbuf_, sbuf_, gsem_, ssem_ = bufs[b]
                w = wo * 2 + b
                g_desc(w, b).wait()

                @pl.when(wo >= 1)
                def _wait_prev():
                    s_desc(w, b).wait()

                for g in range(W // 16):
                    sl = pl.ds(w * W + g * 16, 16)
                    al = (plsc.load_gather(als_v, [src_v[sl]])
                          + plsc.load_gather(ald_v, [dst_v[sl]])
                          + ale_v[sl])
                    ea16 = jnp.exp(jnp.maximum(al, 0.2 * al))
                    ea_v[sl] = ea16
                    for r in range(16):
                        brc = lax.gather(
                            ea16, jnp.full((16, 1), r, jnp.int32), _dn, (1,),
                            mode=lax.GatherScatterMode.PROMISE_IN_BOUNDS)
                        row = g * 16 + r
                        sbuf_[row, :] = gbuf_[row, :] * brc

                @pl.when(wo < NWIN // 2 - 1)
                def _prefetch():
                    gbuf2_, _, gsem2_, _ = bufs[b]
                    pltpu.async_copy(
                        h_s.at[src_v.at[pl.ds((w + 2) * W, W)]],
                        gbuf2_, gsem2_)

                pltpu.async_copy(
                    sbuf_, num_s.at[dst_v.at[pl.ds(w * W, W)]], ssem_,
                    add=True)
            return 0

        lax.fori_loop(0, NWIN // 2, win, 0)
        s_desc(NWIN - 2, 0).wait()
        s_desc(NWIN - 1, 1).wait()
        pltpu.sync_copy(ea_v, den_s.at[dst_v], add=True)
        plsc.subcore_barrier()

        pltpu.sync_copy(num_s.at[pl.ds(s * NT, NT)],
                        num_o.at[c, pl.ds(s * NT, NT)])
        pltpu.sync_copy(den_s.at[pl.ds(s * NT, NT)],
                        den_o.at[c, pl.ds(s * NT, NT)])

    return body(als, ald, h, ale, src, dst)


def kernel(x, edge_index, edge_attr,
           c0_W, c0_as, c0_ad, c0_ae, c0_We, c0_b,
           c1_W, c1_as, c1_ad, c1_ae, c1_We, c1_b,
           c2_W, c2_as, c2_ad, c2_ae, c2_We, c2_b,
           l0_W, l0_b, l1_W, l1_b, l2_W, l2_b, l3_W, l3_b):
    srcp = edge_index[0]
    dstp = edge_index[1]

    we_all = jnp.stack([c0_We, c1_We, c2_We])
    ae_all = jnp.stack([c0_ae, c1_ae, c2_ae])
    edge_attr_p = jnp.pad(edge_attr, ((0, EP - E), (0, 0)))
    ales = list(_edge_logits(edge_attr_p, we_all, ae_all))

    h, als, ald = _node_proj(x, c0_W, c0_as, c0_ad)
    num, den = _sc_edge_layer(als, ald, h, ales[0], srcp, dstp)
    h, als, ald = _combine_proj(num, den, c0_b, c1_W, c1_as, c1_ad)
    num, den = _sc_edge_layer(als, ald, h, ales[1], srcp, dstp)
    h, als, ald = _combine_proj(num, den, c1_b, c2_W, c2_as, c2_ad)
    num, den = _sc_edge_layer(als, ald, h, ales[2], srcp, dstp)
    out = _mlp(num, den, c2_b, l0_W, l0_b, l1_W, l1_b, l2_W, l2_b, l3_W, l3_b)
    return out[:N]


# speedup vs baseline: 1.5319x; 1.5319x over previous
import functools

import jax
import jax.numpy as jnp
from jax import lax
from jax.experimental import pallas as pl
from jax.experimental.pallas import tpu as pltpu
from jax.experimental.pallas import tpu_sc as plsc

N = 10000
NP = 10240
E = 320000
EP = 327680
ER = EP // 128
NC = 2
NS = 16
NW = NC * NS
RT = ER // NW
ETILE = EP // NW
ET_REAL = E // NW
NT = NP // NS
HG = 16
F32 = jnp.float32


def _node_proj_body(x_ref, w_ref, as_ref, ad_ref, h_ref, als_ref, ald_ref):
    h = jnp.dot(x_ref[...], w_ref[...], preferred_element_type=F32)
    h_ref[...] = h
    als_ref[...] = h @ as_ref[...]
    ald_ref[...] = h @ ad_ref[...]


def _node_proj(x, w, a_s, a_d):
    nb = 5
    blk = NP // nb
    d_in = x.shape[1]
    return pl.pallas_call(
        _node_proj_body,
        grid=(nb,),
        in_specs=[
            pl.BlockSpec((blk, d_in), lambda i: (i, 0)),
            pl.BlockSpec((d_in, HG), lambda i: (0, 0)),
            pl.BlockSpec((HG,), lambda i: (0,)),
            pl.BlockSpec((HG,), lambda i: (0,)),
        ],
        out_specs=[
            pl.BlockSpec((blk, HG), lambda i: (i, 0)),
            pl.BlockSpec((blk,), lambda i: (i,)),
            pl.BlockSpec((blk,), lambda i: (i,)),
        ],
        out_shape=[
            jax.ShapeDtypeStruct((NP, HG), F32),
            jax.ShapeDtypeStruct((NP,), F32),
            jax.ShapeDtypeStruct((NP,), F32),
        ],
    )(x, w, a_s, a_d)


def _ale_body(ea_ref, we_ref, ae_ref, o0_ref, o1_ref, o2_ref):
    blk = ea_ref.shape[0]
    ea = ea_ref[...]
    rows = pl.program_id(0) * blk + jax.lax.broadcasted_iota(jnp.int32, (blk,), 0)
    pad_mask = rows >= E
    for i, o_ref in enumerate((o0_ref, o1_ref, o2_ref)):
        v = jnp.dot(we_ref[i], ae_ref[i], preferred_element_type=F32)
        o_ref[...] = jnp.where(pad_mask, -1e30, ea @ v)


def _edge_logits(edge_attr_p, we_all, ae_all):
    nb = 80
    blk = EP // nb
    d_e = edge_attr_p.shape[1]
    return pl.pallas_call(
        _ale_body,
        grid=(nb,),
        in_specs=[
            pl.BlockSpec((blk, d_e), lambda i: (i, 0)),
            pl.BlockSpec((3, d_e, HG), lambda i: (0, 0, 0)),
            pl.BlockSpec((3, HG), lambda i: (0, 0)),
        ],
        out_specs=[pl.BlockSpec((blk,), lambda i: (i,))] * 3,
        out_shape=[jax.ShapeDtypeStruct((EP,), F32)] * 3,
    )(edge_attr_p, we_all, ae_all)


def _combine_proj_body(num_ref, den_ref, b_ref, w_ref, as_ref, ad_ref,
                       h_ref, als_ref, ald_ref):
    num = num_ref[0] + num_ref[1]
    den = den_ref[0] + den_ref[1]
    xn = num / (den[:, None] + 1e-16) + b_ref[...]
    xn = jnp.where(xn >= 0.0, xn, 0.2 * xn)
    h = jnp.dot(xn, w_ref[...], preferred_element_type=F32)
    h_ref[...] = h
    als_ref[...] = h @ as_ref[...]
    ald_ref[...] = h @ ad_ref[...]


def _combine_proj(num, den, b, w, a_s, a_d):
    nb = 5
    blk = NP // nb
    return pl.pallas_call(
        _combine_proj_body,
        grid=(nb,),
        in_specs=[
            pl.BlockSpec((NC, blk, HG), lambda i: (0, i, 0)),
            pl.BlockSpec((NC, blk), lambda i: (0, i)),
            pl.BlockSpec((HG,), lambda i: (0,)),
            pl.BlockSpec((HG, HG), lambda i: (0, 0)),
            pl.BlockSpec((HG,), lambda i: (0,)),
            pl.BlockSpec((HG,), lambda i: (0,)),
        ],
        out_specs=[
            pl.BlockSpec((blk, HG), lambda i: (i, 0)),
            pl.BlockSpec((blk,), lambda i: (i,)),
            pl.BlockSpec((blk,), lambda i: (i,)),
        ],
        out_shape=[
            jax.ShapeDtypeStruct((NP, HG), F32),
            jax.ShapeDtypeStruct((NP,), F32),
            jax.ShapeDtypeStruct((NP,), F32),
        ],
    )(num, den, b, w, a_s, a_d)


def _mlp_body(num_ref, den_ref, b_ref, w0_ref, b0_ref, w1_ref, b1_ref,
              w2_ref, b2_ref, w3_ref, b3_ref, o_ref):
    num = num_ref[0] + num_ref[1]
    den = den_ref[0] + den_ref[1]
    a = num / (den[:, None] + 1e-16) + b_ref[...]
    a = jnp.where(a >= 0.0, a, 0.2 * a)
    for w_r, bias_r in ((w0_ref, b0_ref), (w1_ref, b1_ref), (w2_ref, b2_ref)):
        a = jnp.maximum(
            jnp.dot(a, w_r[...], preferred_element_type=F32) + bias_r[...], 0.0)
    o_ref[...] = jnp.dot(a, w3_ref[...], preferred_element_type=F32) + b3_ref[...]


def _mlp(num, den, b, l0w, l0b, l1w, l1b, l2w, l2b, l3w, l3b):
    nb = 5
    blk = NP // nb
    d_out = l3w.shape[1]
    hl = l0w.shape[1]
    return pl.pallas_call(
        _mlp_body,
        grid=(nb,),
        in_specs=[
            pl.BlockSpec((NC, blk, HG), lambda i: (0, i, 0)),
            pl.BlockSpec((NC, blk), lambda i: (0, i)),
            pl.BlockSpec((HG,), lambda i: (0,)),
            pl.BlockSpec((HG, hl), lambda i: (0, 0)),
            pl.BlockSpec((hl,), lambda i: (0,)),
            pl.BlockSpec((hl, hl), lambda i: (0, 0)),
            pl.BlockSpec((hl,), lambda i: (0,)),
            pl.BlockSpec((hl, hl), lambda i: (0, 0)),
            pl.BlockSpec((hl,), lambda i: (0,)),
            pl.BlockSpec((hl, d_out), lambda i: (0, 0)),
            pl.BlockSpec((d_out,), lambda i: (0,)),
        ],
        out_specs=[pl.BlockSpec((blk, d_out), lambda i: (i, 0))],
        out_shape=[jax.ShapeDtypeStruct((NP, d_out), F32)],
    )(num, den, b, l0w, l0b, l1w, l1b, l2w, l2b, l3w, l3b)[0]


def _sc_edge_layer(als, ald, h, ale, src, dst):
    mesh = plsc.VectorSubcoreMesh(core_axis_name="c", subcore_axis_name="s")
    WR = 2
    W = WR * 128
    NWIN = RT // WR
    _dn = lax.GatherDimensionNumbers(
        offset_dims=(), collapsed_slice_dims=(0,), start_index_map=(0,))

    @functools.partial(
        pl.kernel,
        out_type=[
            jax.ShapeDtypeStruct((NC, NP, HG), F32),
            jax.ShapeDtypeStruct((NC, NP), F32),
        ],
        mesh=mesh,
        scratch_types=[
            pltpu.VMEM((NP,), F32),
            pltpu.VMEM((NP,), F32),
            pltpu.VMEM((ETILE,), jnp.int32),
            pltpu.VMEM((ETILE,), jnp.int32),
            pltpu.VMEM((ETILE,), F32),
            pltpu.VMEM((ETILE,), F32),
            pltpu.VMEM((W, HG), F32),
            pltpu.VMEM((W, HG), F32),
            pltpu.VMEM((W, HG), F32),
            pltpu.VMEM((W, HG), F32),
            pltpu.VMEM((NT, HG), F32),
            pltpu.VMEM((NT,), F32),
            pltpu.VMEM_SHARED((NP, HG), F32),
            pltpu.VMEM_SHARED((NP, HG), F32),
            pltpu.VMEM_SHARED((NP,), F32),
            pltpu.SemaphoreType.DMA,
            pltpu.SemaphoreType.DMA,
            pltpu.SemaphoreType.DMA,
            pltpu.SemaphoreType.DMA,
        ],
        compiler_params=pltpu.CompilerParams(needs_layout_passes=False,
                                             use_tc_tiling_on_sc=False),
    )
    def body(als_h, ald_h, h_h, ale_h, src_h, dst_h, num_o, den_o,
             als_v, ald_v, src_v, dst_v, ale_v, ea_v,
             gbuf0, gbuf1, sbuf0, sbuf1, znum_v, zden_v,
             h_s, num_s, den_s, gsem0, gsem1, ssem0, ssem1):
        c = lax.axis_index("c")
        s = lax.axis_index("s")
        wid = s * NC + c
        bufs = ((gbuf0, sbuf0, gsem0, ssem0), (gbuf1, sbuf1, gsem1, ssem1))

        pltpu.sync_copy(als_h, als_v)
        pltpu.sync_copy(ald_h, ald_v)
        pltpu.sync_copy(h_h.at[pl.ds(s * NT, NT)], h_s.at[pl.ds(s * NT, NT)])
        zv = jnp.zeros((16,), F32)

        def zrow(j, _):
            znum_v[j, :] = zv
            return 0

        lax.fori_loop(0, NT, zrow, 0)

        def zden(j, _):
            zden_v[pl.ds(j * 16, 16)] = zv
            return 0

        lax.fori_loop(0, NT // 16, zden, 0)
        pltpu.sync_copy(znum_v, num_s.at[pl.ds(s * NT, NT)])
        pltpu.sync_copy(zden_v, den_s.at[pl.ds(s * NT, NT)])

        pltpu.sync_copy(src_h.at[pl.ds(wid * ET_REAL, ET_REAL)],
                        src_v.at[pl.ds(0, ET_REAL)])
        pltpu.sync_copy(dst_h.at[pl.ds(wid * ET_REAL, ET_REAL)],
                        dst_v.at[pl.ds(0, ET_REAL)])
        pltpu.sync_copy(ale_h.at[pl.ds(wid * ET_REAL, ET_REAL)],
                        ale_v.at[pl.ds(0, ET_REAL)])
        iv16 = lax.broadcasted_iota(jnp.int32, (16,), 0)

        def ptail(j, _):
            sl = pl.ds(ET_REAL + j * 16, 16)
            pad_ix = (iv16 * 977 + j * 131) % N
            src_v[sl] = pad_ix
            dst_v[sl] = pad_ix
            ale_v[sl] = jnp.full((16,), -1e30, F32)
            return 0

        lax.fori_loop(0, (ETILE - ET_REAL) // 16, ptail, 0)
        plsc.subcore_barrier()

        def g_desc(w, b):
            gbuf_, _, gsem_, _ = bufs[b]
            return pltpu.make_async_copy(
                h_s.at[src_v.at[pl.ds(w * W, W)]], gbuf_, gsem_)

        def s_desc(w, b):
            _, sbuf_, _, ssem_ = bufs[b]
            return pltpu.make_async_copy(
                sbuf_, num_s.at[dst_v.at[pl.ds(w * W, W)]], ssem_)

        pltpu.async_copy(h_s.at[src_v.at[pl.ds(0, W)]], gbuf0, gsem0)
        pltpu.async_copy(h_s.at[src_v.at[pl.ds(W, W)]], gbuf1, gsem1)

        def win(wo, _):
            for b in range(2):
                gbuf_, sbuf_, gsem_, ssem_ = bufs[b]
                w = wo * 2 + b
                g_desc(w, b).wait()

                @pl.when(wo >= 1)
                def _wait_prev():
                    s_desc(w, b).wait()

                for g in range(W // 16):
                    sl = pl.ds(w * W + g * 16, 16)
                    al = (plsc.load_gather(als_v, [src_v[sl]])
                          + plsc.load_gather(ald_v, [dst_v[sl]])
                          + ale_v[sl])
                    ea16 = jnp.exp(jnp.maximum(al, 0.2 * al))
                    ea_v[sl] = ea16
                    for r in range(16):
                        brc = lax.gather(
                            ea16, jnp.full((16, 1), r, jnp.int32), _dn, (1,),
                            mode=lax.GatherScatterMode.PROMISE_IN_BOUNDS)
                        row = g * 16 + r
                        sbuf_[row, :] = gbuf_[row, :] * brc

                @pl.when(wo < NWIN // 2 - 1)
                def _prefetch():
                    gbuf2_, _, gsem2_, _ = bufs[b]
                    pltpu.async_copy(
                        h_s.at[src_v.at[pl.ds((w + 2) * W, W)]],
                        gbuf2_, gsem2_)

                pltpu.async_copy(
                    sbuf_, num_s.at[dst_v.at[pl.ds(w * W, W)]], ssem_,
                    add=True)
            return 0

        lax.fori_loop(0, NWIN // 2, win, 0)
        s_desc(NWIN - 2, 0).wait()
        s_desc(NWIN - 1, 1).wait()
        pltpu.sync_copy(ea_v, den_s.at[dst_v], add=True)
        plsc.subcore_barrier()

        pltpu.sync_copy(num_s.at[pl.ds(s * NT, NT)],
                        num_o.at[c, pl.ds(s * NT, NT)])
        pltpu.sync_copy(den_s.at[pl.ds(s * NT, NT)],
                        den_o.at[c, pl.ds(s * NT, NT)])

    return body(als, ald, h, ale, src, dst)


def kernel(x, edge_index, edge_attr,
           c0_W, c0_as, c0_ad, c0_ae, c0_We, c0_b,
           c1_W, c1_as, c1_ad, c1_ae, c1_We, c1_b,
           c2_W, c2_as, c2_ad, c2_ae, c2_We, c2_b,
           l0_W, l0_b, l1_W, l1_b, l2_W, l2_b, l3_W, l3_b):
    srcp = edge_index[0]
    dstp = edge_index[1]

    we_all = jnp.stack([c0_We, c1_We, c2_We])
    ae_all = jnp.stack([c0_ae, c1_ae, c2_ae])
    edge_attr_p = jnp.pad(edge_attr, ((0, EP - E), (0, 0)))
    ales = list(_edge_logits(edge_attr_p, we_all, ae_all))

    h, als, ald = _node_proj(x, c0_W, c0_as, c0_ad)
    num, den = _sc_edge_layer(als, ald, h, ales[0], srcp, dstp)
    h, als, ald = _combine_proj(num, den, c0_b, c1_W, c1_as, c1_ad)
    num, den = _sc_edge_layer(als, ald, h, ales[1], srcp, dstp)
    h, als, ald = _combine_proj(num, den, c1_b, c2_W, c2_as, c2_ad)
    num, den = _sc_edge_layer(als, ald, h, ales[2], srcp, dstp)
    out = _mlp(num, den, c2_b, l0_W, l0_b, l1_W, l1_b, l2_W, l2_b, l3_W, l3_b)
    return out[:N]
